# baseline (speedup 1.0000x reference)
import functools

import jax
import jax.numpy as jnp
from jax import lax
from jax.experimental import pallas as pl
from jax.experimental.pallas import tpu as pltpu
from jax.experimental.pallas import tpu_sc as plsc

L = 16
NC = 2
NS = 16
GB = 128
NBUF = 4
BGB = 8
OC = 1024
BBR = 32


def kernel(flat_features, offsets, weight, bias):
    N = flat_features.shape[0]
    B = offsets.shape[0]
    V, D = weight.shape
    HD = D // 2
    CH = N // NS
    CH2 = CH // 2
    NB2 = CH2 // GB
    NBG = NB2 // BGB
    BT = B // NS

    table2 = weight.reshape(2 * V, HD)
    ff2 = flat_features.reshape(N // GB, GB)
    bias2 = bias.reshape(NC, HD)

    mesh = plsc.VectorSubcoreMesh(
        core_axis_name="c", subcore_axis_name="s", num_cores=NC, num_subcores=NS
    )

    scratch = [
        pltpu.VMEM_SHARED((B, HD), jnp.float32),
        pltpu.VMEM((CH2,), jnp.int32),
        pltpu.VMEM((NBUF, GB, HD), jnp.float32),
    ]
    scratch += [pltpu.VMEM((BGB, GB), jnp.int32) for _ in range(2)]
    scratch += [pltpu.VMEM((GB,), jnp.int32) for _ in range(NBUF)]
    scratch += [
        pltpu.VMEM((OC,), jnp.int32),
        pltpu.VMEM((BBR, HD), jnp.float32),
        pltpu.VMEM((HD,), jnp.float32),
    ]
    scratch += [pltpu.SemaphoreType.DMA for _ in range(2 + 2 * NBUF)]

    @functools.partial(
        pl.kernel,
        out_type=jax.ShapeDtypeStruct((B, D), jnp.float32),
        mesh=mesh,
        scratch_types=scratch,
        compiler_params=pltpu.CompilerParams(
            use_tc_tiling_on_sc=False, needs_layout_passes=False),
    )
    def run(ff_h, off_h, tab_h, bias_h, out_h, acc, zcnt, rows, *rest):
        ibig = rest[0:2]
        segb = rest[2:2 + NBUF]
        offc, bb, bt = rest[2 + NBUF:5 + NBUF]
        isem = rest[5 + NBUF:7 + NBUF]
        gsem = rest[7 + NBUF:7 + 2 * NBUF]
        ssem = rest[7 + 2 * NBUF:7 + 3 * NBUF]

        c = lax.axis_index("c")
        sid = lax.axis_index("s")
        pos0 = sid * CH

        zero16 = jnp.zeros((L,), jnp.int32)
        one16 = jnp.ones((L,), jnp.int32)

        pltpu.sync_copy(bias_h.at[c], bt)

        def bb_fill(r, _):
            for u in range(HD // L):
                bb[r, pl.ds(u * L, L)] = bt[pl.ds(u * L, L)]
            return 0
        lax.fori_loop(0, BBR, bb_fill, 0)

        for k in range(BT // BBR):
            pltpu.async_copy(
                bb, acc.at[pl.ds(sid * BT + k * BBR, BBR)], isem[0])
        for k in range(BT // BBR):
            pltpu.make_async_copy(
                bb, acc.at[pl.ds(sid * BT + k * BBR, BBR)], isem[0]).wait()

        plsc.subcore_barrier()

        for h in range(2):
            h0 = pos0 + h * CH2
            row0 = h0 // GB

            def z0(k, _):
                for u in range(4):
                    zcnt[pl.ds((k * 4 + u) * L, L)] = zero16
                return 0
            lax.fori_loop(0, CH2 // L // 4, z0, 0)

            def ccount(kc, base):
                pltpu.sync_copy(off_h.at[pl.ds(kc * OC, OC)], offc)
                first = jnp.min(offc[pl.ds(0, L)])
                last = jnp.max(offc[pl.ds(OC - L, L)])

                def sweep():
                    def inner(kv, cv):
                        o = offc[pl.ds(kv * L, L)]
                        m = (o >= h0) & (o < h0 + CH2)
                        plsc.addupdate_scatter(zcnt, [o - h0], one16, mask=m)
                        return cv + jnp.where(o < h0, one16, zero16)
                    cntv = lax.fori_loop(0, OC // L, inner, zero16)
                    return base + jnp.sum(cntv)
                return lax.cond(
                    last < h0, lambda: base + OC,
                    lambda: lax.cond(first >= h0 + CH2, lambda: base, sweep))
            base = lax.fori_loop(0, B // OC, ccount, jnp.int32(0))

            def scan_body(k, carry):
                for u in range(4):
                    kk = k * 4 + u
                    v = zcnt[pl.ds(kk * L, L)]

                    def nz_path(carry=carry, kk=kk, v=v):
                        zcnt[pl.ds(kk * L, L)] = plsc.cumsum(v) + carry
                        return carry + jnp.sum(v)

                    def z_path(carry=carry, kk=kk):
                        zcnt[pl.ds(kk * L, L)] = carry
                        return carry
                    carry = lax.cond(jnp.any(v != 0), nz_path, z_path)
                return carry
            lax.fori_loop(0, CH2 // L // 4, scan_body,
                          jnp.full((L,), -1, jnp.int32) + base)

            pltpu.async_copy(ff_h.at[pl.ds(row0, BGB)], ibig[0], isem[0])

            def big_group(g, par, tail):
                ib = ibig[par]
                pltpu.make_async_copy(
                    ff_h.at[pl.ds(row0 + g * BGB, BGB)], ib, isem[par]).wait()
                if not tail:
                    @pl.when(g + 1 < NBG)
                    def _pf():
                        pltpu.async_copy(
                            ff_h.at[pl.ds(row0 + (g + 1) * BGB, BGB)],
                            ibig[1 - par], isem[1 - par])

                for r in range(BGB // NBUF):
                    for s in range(NBUF):
                        half_slot = r * NBUF + s
                        lb = (g * BGB + half_slot) * GB

                        if r == 0 and not tail:
                            @pl.when(g > 0)
                            def _w(s=s):
                                pltpu.make_async_copy(
                                    rows.at[s], acc.at[segb[s]],
                                    ssem[s]).wait()
                        else:
                            pltpu.make_async_copy(
                                rows.at[s], acc.at[segb[s]], ssem[s]).wait()

                        for u in range(GB // L):
                            v = ib[half_slot, pl.ds(u * L, L)]
                            ib[half_slot, pl.ds(u * L, L)] = v * 2 + c
                            segb[s][pl.ds(u * L, L)] = \
                                zcnt[pl.ds(lb + u * L, L)]
                        pltpu.async_copy(
                            tab_h.at[ib.at[half_slot]], rows.at[s], gsem[s])

                    for s in range(NBUF):
                        half_slot = r * NBUF + s
                        pltpu.make_async_copy(
                            tab_h.at[ib.at[half_slot]], rows.at[s],
                            gsem[s]).wait()
                        pltpu.async_copy(
                            rows.at[s], acc.at[segb[s]], ssem[s], add=True)
                return 0

            def pair(j, _):
                big_group(2 * j, 0, False)
                big_group(2 * j + 1, 1, False)
                return 0
            lax.fori_loop(0, NBG // 2, pair, 0)
            big_group(NBG - 1, (NBG - 1) % 2, True)

            for s in range(NBUF):
                pltpu.make_async_copy(
                    rows.at[s], acc.at[segb[s]], ssem[s]).wait()

        plsc.subcore_barrier()

        pltpu.sync_copy(
            acc.at[pl.ds(sid * BT, BT)],
            out_h.at[pl.ds(sid * BT, BT), pl.ds(c * HD, HD)])

    return run(ff2, offsets, table2, bias2)


# speedup vs baseline: 1.0416x; 1.0416x over previous
import functools

import jax
import jax.numpy as jnp
from jax import lax
from jax.experimental import pallas as pl
from jax.experimental.pallas import tpu as pltpu
from jax.experimental.pallas import tpu_sc as plsc

L = 16
NC = 2
NS = 16
GB = 128
NBUF = 4
BGB = 8
OC = 1024
BBR = 32


def kernel(flat_features, offsets, weight, bias):
    N = flat_features.shape[0]
    B = offsets.shape[0]
    V, D = weight.shape
    HD = D // 2
    CH = N // NS
    CH2 = CH // 2
    NB2 = CH2 // GB
    NBG = NB2 // BGB
    BT = B // NS

    table2 = weight.reshape(2 * V, HD)
    ff2 = flat_features.reshape(N // GB, GB)
    bias2 = bias.reshape(NC, HD)

    mesh = plsc.VectorSubcoreMesh(
        core_axis_name="c", subcore_axis_name="s", num_cores=NC, num_subcores=NS
    )

    scratch = [
        pltpu.VMEM_SHARED((B, HD), jnp.float32),
        pltpu.VMEM((CH2,), jnp.int32),
        pltpu.VMEM((NBUF, GB, HD), jnp.float32),
    ]
    scratch += [pltpu.VMEM((BGB, GB), jnp.int32) for _ in range(2)]
    scratch += [pltpu.VMEM((GB,), jnp.int32) for _ in range(NBUF)]
    scratch += [
        pltpu.VMEM((OC,), jnp.int32),
        pltpu.VMEM((BBR, HD), jnp.float32),
        pltpu.VMEM((HD,), jnp.float32),
    ]
    scratch += [pltpu.SemaphoreType.DMA for _ in range(2 + 2 * NBUF)]

    @functools.partial(
        pl.kernel,
        out_type=jax.ShapeDtypeStruct((B, D), jnp.float32),
        mesh=mesh,
        scratch_types=scratch,
        compiler_params=pltpu.CompilerParams(
            use_tc_tiling_on_sc=False, needs_layout_passes=False),
    )
    def run(ff_h, off_h, tab_h, bias_h, out_h, acc, zcnt, rows, *rest):
        ibig = rest[0:2]
        segb = rest[2:2 + NBUF]
        offc, bb, bt = rest[2 + NBUF:5 + NBUF]
        isem = rest[5 + NBUF:7 + NBUF]
        gsem = rest[7 + NBUF:7 + 2 * NBUF]
        ssem = rest[7 + 2 * NBUF:7 + 3 * NBUF]

        c = lax.axis_index("c")
        sid = lax.axis_index("s")
        pos0 = sid * CH

        zero16 = jnp.zeros((L,), jnp.int32)
        one16 = jnp.ones((L,), jnp.int32)

        pltpu.sync_copy(bias_h.at[c], bt)

        def bb_fill(r, _):
            for u in range(HD // L):
                bb[r, pl.ds(u * L, L)] = bt[pl.ds(u * L, L)]
            return 0
        lax.fori_loop(0, BBR, bb_fill, 0)

        for k in range(BT // BBR):
            pltpu.async_copy(
                bb, acc.at[pl.ds(sid * BT + k * BBR, BBR)], isem[0])
        for k in range(BT // BBR):
            pltpu.make_async_copy(
                bb, acc.at[pl.ds(sid * BT + k * BBR, BBR)], isem[0]).wait()

        plsc.subcore_barrier()

        for h in range(2):
            h0 = pos0 + h * CH2
            row0 = h0 // GB

            def z0(k, _):
                for u in range(4):
                    zcnt[pl.ds((k * 4 + u) * L, L)] = zero16
                return 0
            lax.fori_loop(0, CH2 // L // 4, z0, 0)

            def ccount(kc, base):
                pltpu.sync_copy(off_h.at[pl.ds(kc * OC, OC)], offc)
                first = jnp.min(offc[pl.ds(0, L)])
                last = jnp.max(offc[pl.ds(OC - L, L)])

                def sweep():
                    def inner(kv, cv):
                        o = offc[pl.ds(kv * L, L)]
                        m = (o >= h0) & (o < h0 + CH2)
                        plsc.addupdate_scatter(zcnt, [o - h0], one16, mask=m)
                        return cv + jnp.where(o < h0, one16, zero16)
                    cntv = lax.fori_loop(0, OC // L, inner, zero16)
                    return base + jnp.sum(cntv)
                return lax.cond(
                    last < h0, lambda: base + OC,
                    lambda: lax.cond(first >= h0 + CH2, lambda: base, sweep))
            base = lax.fori_loop(0, B // OC, ccount, jnp.int32(0))

            def scan_body(k, carry):
                for u in range(4):
                    kk = k * 4 + u
                    v = zcnt[pl.ds(kk * L, L)]
                    zcnt[pl.ds(kk * L, L)] = plsc.cumsum(v) + carry
                    carry = carry + jnp.sum(v)
                return carry
            lax.fori_loop(0, CH2 // L // 4, scan_body,
                          jnp.full((L,), -1, jnp.int32) + base)

            pltpu.async_copy(ff_h.at[pl.ds(row0, BGB)], ibig[0], isem[0])

            def big_group(g, par, tail):
                ib = ibig[par]
                pltpu.make_async_copy(
                    ff_h.at[pl.ds(row0 + g * BGB, BGB)], ib, isem[par]).wait()
                if not tail:
                    @pl.when(g + 1 < NBG)
                    def _pf():
                        pltpu.async_copy(
                            ff_h.at[pl.ds(row0 + (g + 1) * BGB, BGB)],
                            ibig[1 - par], isem[1 - par])

                for r in range(BGB // NBUF):
                    for s in range(NBUF):
                        half_slot = r * NBUF + s
                        lb = (g * BGB + half_slot) * GB

                        if r == 0 and not tail:
                            @pl.when(g > 0)
                            def _w(s=s):
                                pltpu.make_async_copy(
                                    rows.at[s], acc.at[segb[s]],
                                    ssem[s]).wait()
                        else:
                            pltpu.make_async_copy(
                                rows.at[s], acc.at[segb[s]], ssem[s]).wait()

                        for u in range(GB // L):
                            v = ib[half_slot, pl.ds(u * L, L)]
                            ib[half_slot, pl.ds(u * L, L)] = v * 2 + c
                            segb[s][pl.ds(u * L, L)] = \
                                zcnt[pl.ds(lb + u * L, L)]
                        pltpu.async_copy(
                            tab_h.at[ib.at[half_slot]], rows.at[s], gsem[s])

                    for s in range(NBUF):
                        half_slot = r * NBUF + s
                        pltpu.make_async_copy(
                            tab_h.at[ib.at[half_slot]], rows.at[s],
                            gsem[s]).wait()
                        pltpu.async_copy(
                            rows.at[s], acc.at[segb[s]], ssem[s], add=True)
                return 0

            def pair(j, _):
                big_group(2 * j, 0, False)
                big_group(2 * j + 1, 1, False)
                return 0
            lax.fori_loop(0, NBG // 2, pair, 0)
            big_group(NBG - 1, (NBG - 1) % 2, True)

            for s in range(NBUF):
                pltpu.make_async_copy(
                    rows.at[s], acc.at[segb[s]], ssem[s]).wait()

        plsc.subcore_barrier()

        pltpu.sync_copy(
            acc.at[pl.ds(sid * BT, BT)],
            out_h.at[pl.ds(sid * BT, BT), pl.ds(c * HD, HD)])

    return run(ff2, offsets, table2, bias2)
